# Initial kernel scaffold; baseline (speedup 1.0000x reference)
import jax
import jax.numpy as jnp
from jax.experimental import pallas as pl


def kernel(x, e, g, edges, node_idx, edge_idx, W_edge, b_edge, W_node, b_node, W_glob, b_glob):
    raise NotImplementedError("write your pallas kernel here")


# speedup vs baseline: 7.4439x; 7.4439x over previous
import functools

import jax
import jax.numpy as jnp
from jax import lax
from jax.experimental import pallas as pl
from jax.experimental.pallas import tpu as pltpu
from jax.experimental.pallas import tpu_sc as plsc

NC = 2
NS = 16
NW = NC * NS
LANES = 16


def _row_proj_body(w_ref, b_ref, x_ref, out_ref):
    out_ref[...] = lax.dot_general(
        w_ref[...], x_ref[...], (((1,), (1,)), ((), ())),
        preferred_element_type=jnp.float32,
        precision=lax.Precision.HIGHEST) + b_ref[...]


def _row_proj(w8, b8, arr, blk):
    n, d = arr.shape
    assert n % blk == 0
    return pl.pallas_call(
        _row_proj_body,
        grid=(n // blk,),
        in_specs=[
            pl.BlockSpec((8, d), lambda i: (0, 0)),
            pl.BlockSpec((8, 1), lambda i: (0, 0)),
            pl.BlockSpec((blk, d), lambda i: (i, 0)),
        ],
        out_specs=pl.BlockSpec((8, blk), lambda i: (0, i)),
        out_shape=jax.ShapeDtypeStruct((8, n), jnp.float32),
    )(w8, b8, arr)


def _sc_edge_body(ps_hbm, pd_hbm, pge_hbm, nid_hbm, pe_hbm, src_hbm, dst_hbm,
                  eidx_hbm, zeros_hbm,
                  eo_hbm, e2n_hbm, eg_hbm, ng_hbm,
                  ps_v, pd_v, nid_v, acc_v, pge_v, eg_v, ng_v,
                  src_v, dst_v, eidx_v, pe_v, eo_v,
                  npad, eb, nb):
    c = lax.axis_index("c")
    s = lax.axis_index("s")
    wid = s * NC + c
    base = wid * eb
    pltpu.sync_copy(ps_hbm, ps_v)
    pltpu.sync_copy(pd_hbm, pd_v)
    pltpu.sync_copy(nid_hbm, nid_v)
    pltpu.sync_copy(pge_hbm, pge_v)
    pltpu.sync_copy(zeros_hbm, acc_v)
    pltpu.sync_copy(src_hbm.at[pl.ds(base, eb)], src_v)
    pltpu.sync_copy(dst_hbm.at[pl.ds(base, eb)], dst_v)
    pltpu.sync_copy(eidx_hbm.at[pl.ds(base, eb)], eidx_v)
    pltpu.sync_copy(pe_hbm.at[pl.ds(base, eb)], pe_v)
    zero16 = jnp.zeros((LANES,), jnp.float32)
    for i in range(0, 128, LANES):
        eg_v[pl.ds(i, LANES)] = zero16
        ng_v[pl.ds(i, LANES)] = zero16

    def body(j, carry):
        o = j * LANES
        s16 = src_v[pl.ds(o, LANES)]
        d16 = dst_v[pl.ds(o, LANES)]
        g16 = eidx_v[pl.ds(o, LANES)]
        val = (pe_v[pl.ds(o, LANES)]
               + plsc.load_gather(ps_v, [s16])
               + plsc.load_gather(pd_v, [d16])
               + plsc.load_gather(pge_v, [g16]))
        eo_v[pl.ds(o, LANES)] = val
        plsc.addupdate_scatter(acc_v, [d16], val)
        plsc.addupdate_scatter(eg_v, [g16], val)
        nd16 = plsc.load_gather(nid_v, [d16])
        plsc.addupdate_scatter(ng_v, [nd16], val)
        return carry

    lax.fori_loop(0, nb, body, 0)
    pltpu.sync_copy(eo_v, eo_hbm.at[pl.ds(base, eb)])
    pltpu.sync_copy(acc_v, e2n_hbm.at[wid])
    pltpu.sync_copy(eg_v, eg_hbm.at[wid])
    pltpu.sync_copy(ng_v, ng_hbm.at[wid])


def _finish_body(parts_ref, pxnb_ref, nid_ref, gproj_ref, egp_ref, ngp_ref,
                 scal_ref, node_ref, glob_ref, n_rows, n_glob):
    wn_a = scal_ref[0]
    wg_n = scal_ref[1]
    wg_e = scal_ref[2]
    e2n = parts_ref[0:n_rows, :]
    for w in range(1, NW):
        e2n = e2n + parts_ref[w * n_rows:(w + 1) * n_rows, :]
    nid = nid_ref[...]
    pxnb = pxnb_ref[...]
    pgn_full = jnp.zeros_like(pxnb)
    for sg in range(n_glob):
        pgn_full = pgn_full + jnp.where(nid == sg, gproj_ref[1, sg], 0.0)
    node_out = pxnb + pgn_full + wn_a * e2n
    node_ref[...] = node_out
    lane = lax.broadcasted_iota(jnp.int32, (1, 128), 1)
    base = jnp.zeros((1, 128), jnp.float32)
    for sg in range(n_glob):
        m = nid == sg
        val = jnp.sum(jnp.where(m, pxnb, 0.0)) + gproj_ref[1, sg] * jnp.sum(
            m.astype(jnp.float32))
        base = base + jnp.where(lane == sg, val, 0.0)
    n2g = base + wn_a * jnp.sum(ngp_ref[...], axis=0, keepdims=True)
    e2g = jnp.sum(egp_ref[...], axis=0, keepdims=True)
    glob = (gproj_ref[2:3, 0:n_glob] + wg_n * n2g[:, 0:n_glob]
            + wg_e * e2g[:, 0:n_glob])
    glob_ref[...] = jnp.broadcast_to(glob, (8, n_glob))


def kernel(x, e, g, edges, node_idx, edge_idx,
           W_edge, b_edge, W_node, b_node, W_glob, b_glob):
    n, dn = x.shape
    ne, de = e.shape
    ng, dg = g.shape
    npad = ((n + 2047) // 2048) * 2048
    epad = ((ne + 8191) // 8192) * 8192
    assert epad % (NW * LANES) == 0 and npad % 128 == 0
    eb = epad // NW
    nb = eb // LANES
    n_rows = npad // 128

    f32 = jnp.float32
    w_x = jnp.zeros((8, dn), f32)
    w_x = w_x.at[0].set(W_edge[0, de:de + dn])
    w_x = w_x.at[1].set(W_edge[0, de + dn:de + 2 * dn])
    w_x = w_x.at[2].set(W_node[0, :dn])
    b_x = jnp.zeros((8, 1), f32).at[2].set(b_node[0])
    w_e = jnp.zeros((8, de), f32).at[0].set(W_edge[0, :de])
    b_e = jnp.zeros((8, 1), f32).at[0].set(b_edge[0])
    w_g = jnp.zeros((8, dg), f32)
    w_g = w_g.at[0].set(W_edge[0, de + 2 * dn:])
    w_g = w_g.at[1].set(W_node[0, dn:dn + dg])
    w_g = w_g.at[2].set(W_glob[0, :dg])
    b_g = jnp.zeros((8, 1), f32).at[2].set(b_glob[0])

    x_pad = jnp.zeros((npad, dn), f32).at[:n].set(x)
    e_pad = jnp.zeros((epad, de), f32).at[:ne].set(e)

    px = _row_proj(w_x, b_x, x_pad, 2048)
    pe8 = _row_proj(w_e, b_e, e_pad, 8192)
    pg8 = _row_proj(w_g, b_g, g, ng)

    i32 = jnp.int32
    srcp = jnp.full((epad,), 0, i32).at[:ne].set(edges[0])
    dstp = jnp.full((epad,), n, i32).at[:ne].set(edges[1])
    eidxp = jnp.full((epad,), ng, i32).at[:ne].set(edge_idx)
    nidp = jnp.full((npad,), ng, i32).at[:n].set(node_idx)
    pge32 = jnp.zeros((128,), f32).at[:ng].set(pg8[0])
    zeros_n = jnp.zeros((npad,), f32)

    mesh = plsc.VectorSubcoreMesh(core_axis_name="c", subcore_axis_name="s",
                                  num_cores=NC, num_subcores=NS)
    sc_edge = pl.kernel(
        functools.partial(_sc_edge_body, npad=npad, eb=eb, nb=nb),
        out_type=[
            jax.ShapeDtypeStruct((epad,), f32),
            jax.ShapeDtypeStruct((NW, npad), f32),
            jax.ShapeDtypeStruct((NW, 128), f32),
            jax.ShapeDtypeStruct((NW, 128), f32),
        ],
        mesh=mesh,
        compiler_params=pltpu.CompilerParams(needs_layout_passes=False),
        scratch_types=[
            pltpu.VMEM((npad,), f32),
            pltpu.VMEM((npad,), f32),
            pltpu.VMEM((npad,), i32),
            pltpu.VMEM((npad,), f32),
            pltpu.VMEM((128,), f32),
            pltpu.VMEM((128,), f32),
            pltpu.VMEM((128,), f32),
            pltpu.VMEM((eb,), i32),
            pltpu.VMEM((eb,), i32),
            pltpu.VMEM((eb,), i32),
            pltpu.VMEM((eb,), f32),
            pltpu.VMEM((eb,), f32),
        ],
    )
    eo_pad, e2n_parts, eg_parts, ng_parts = sc_edge(
        px[0], px[1], pge32, nidp, pe8[0], srcp, dstp, eidxp, zeros_n)

    scal = jnp.stack([W_node[0, dn + dg], W_glob[0, dg], W_glob[0, dg + 1],
                      jnp.float32(0)])
    node2, glob8 = pl.pallas_call(
        functools.partial(_finish_body, n_rows=n_rows, n_glob=ng),
        in_specs=[
            pl.BlockSpec(memory_space=pltpu.VMEM),
            pl.BlockSpec(memory_space=pltpu.VMEM),
            pl.BlockSpec(memory_space=pltpu.VMEM),
            pl.BlockSpec(memory_space=pltpu.VMEM),
            pl.BlockSpec(memory_space=pltpu.VMEM),
            pl.BlockSpec(memory_space=pltpu.VMEM),
            pl.BlockSpec(memory_space=pltpu.SMEM),
        ],
        out_shape=[
            jax.ShapeDtypeStruct((n_rows, 128), f32),
            jax.ShapeDtypeStruct((8, ng), f32),
        ],
    )(e2n_parts.reshape(NW * n_rows, 128), px[2].reshape(n_rows, 128),
      nidp.reshape(n_rows, 128), pg8, eg_parts, ng_parts, scal)

    edge_out = eo_pad[:ne, None]
    node_out = node2.reshape(npad)[:n, None]
    glob_out = glob8[0][:, None]
    return (edge_out, node_out, glob_out)


# speedup vs baseline: 8.5038x; 1.1424x over previous
import functools

import jax
import jax.numpy as jnp
from jax import lax
from jax.experimental import pallas as pl
from jax.experimental.pallas import tpu as pltpu
from jax.experimental.pallas import tpu_sc as plsc

NC = 2
NS = 16
NW = NC * NS
LANES = 16


def _xproj_body(w_ref, x_ref, ps_ref, pd_ref, px_ref):
    xblk = x_ref[...]
    ps_ref[...] = jnp.sum(xblk * w_ref[0:1, :], axis=1)
    pd_ref[...] = jnp.sum(xblk * w_ref[1:2, :], axis=1)
    px_ref[...] = jnp.sum(xblk * w_ref[2:3, :], axis=1)


def _eproj_body(w_ref, e_ref, pe_ref):
    pe_ref[...] = jnp.sum(e_ref[...] * w_ref[0:1, :], axis=1)


def _gproj_body(w_ref, b_ref, g_ref, out_ref):
    out_ref[...] = lax.dot_general(
        w_ref[...], g_ref[...], (((1,), (1,)), ((), ())),
        preferred_element_type=jnp.float32,
        precision=lax.Precision.HIGHEST) + b_ref[...]


def _sc_stage_chunk(pe_hbm, src_hbm, dst_hbm, eidx_hbm,
                    src_v, dst_v, eidx_v, pe_v, base, eb):
    pltpu.sync_copy(src_hbm.at[pl.ds(base, eb)], src_v.at[pl.ds(0, eb)])
    pltpu.sync_copy(dst_hbm.at[pl.ds(base, eb)], dst_v.at[pl.ds(0, eb)])
    pltpu.sync_copy(eidx_hbm.at[pl.ds(base, eb)], eidx_v.at[pl.ds(0, eb)])
    pltpu.sync_copy(pe_hbm.at[pl.ds(base, eb)], pe_v.at[pl.ds(0, eb)])


def _sc_edge_body(ps_hbm, pd_hbm, pge_hbm, nid_hbm, pe_hbm, src_hbm, dst_hbm,
                  eidx_hbm, zeros_hbm,
                  eo_hbm, e2n_hbm, eg_hbm, ng_hbm,
                  ps_v, pd_v, nid_v, acc_v, pge_v, eg_v, ng_v,
                  src_v, dst_v, eidx_v, pe_v, eo_v,
                  nv_lo, extra):
    c = lax.axis_index("c")
    s = lax.axis_index("s")
    wid = s * NC + c
    is_hi = wid < extra
    base = jnp.where(is_hi, wid * (nv_lo + 1), wid * nv_lo + extra) * LANES
    pltpu.sync_copy(ps_hbm, ps_v)
    pltpu.sync_copy(pd_hbm, pd_v)
    pltpu.sync_copy(nid_hbm, nid_v)
    pltpu.sync_copy(pge_hbm, pge_v)
    pltpu.sync_copy(zeros_hbm, acc_v)
    eb_hi = (nv_lo + 1) * LANES
    eb_lo = nv_lo * LANES
    pl.when(is_hi)(lambda: _sc_stage_chunk(
        pe_hbm, src_hbm, dst_hbm, eidx_hbm,
        src_v, dst_v, eidx_v, pe_v, base, eb_hi))
    pl.when(jnp.logical_not(is_hi))(lambda: _sc_stage_chunk(
        pe_hbm, src_hbm, dst_hbm, eidx_hbm,
        src_v, dst_v, eidx_v, pe_v, base, eb_lo))
    zero16 = jnp.zeros((LANES,), jnp.float32)
    for i in range(0, 128, LANES):
        eg_v[pl.ds(i, LANES)] = zero16
        ng_v[pl.ds(i, LANES)] = zero16

    nvec = jnp.where(is_hi, nv_lo + 1, nv_lo)

    def body(j, carry):
        o = j * LANES
        s16 = src_v[pl.ds(o, LANES)]
        d16 = dst_v[pl.ds(o, LANES)]
        g16 = eidx_v[pl.ds(o, LANES)]
        val = (pe_v[pl.ds(o, LANES)]
               + plsc.load_gather(ps_v, [s16])
               + plsc.load_gather(pd_v, [d16])
               + plsc.load_gather(pge_v, [g16]))
        eo_v[pl.ds(o, LANES)] = val
        plsc.addupdate_scatter(acc_v, [d16], val)
        plsc.addupdate_scatter(eg_v, [g16], val)
        nd16 = plsc.load_gather(nid_v, [d16])
        plsc.addupdate_scatter(ng_v, [nd16], val)
        return carry

    lax.fori_loop(0, nvec, body, 0)
    pl.when(is_hi)(lambda: pltpu.sync_copy(
        eo_v.at[pl.ds(0, eb_hi)], eo_hbm.at[pl.ds(base, eb_hi)]))
    pl.when(jnp.logical_not(is_hi))(lambda: pltpu.sync_copy(
        eo_v.at[pl.ds(0, eb_lo)], eo_hbm.at[pl.ds(base, eb_lo)]))
    pltpu.sync_copy(acc_v, e2n_hbm.at[wid])
    pltpu.sync_copy(eg_v, eg_hbm.at[wid])
    pltpu.sync_copy(ng_v, ng_hbm.at[wid])


def _finish_body(parts_ref, pxnb_ref, nid_ref, gproj_ref, egp_ref, ngp_ref,
                 scal_ref, node_ref, glob_ref, n_rows, n_glob):
    wn_a = scal_ref[0]
    wg_n = scal_ref[1]
    wg_e = scal_ref[2]
    b_n = scal_ref[3]
    e2n = parts_ref[0:n_rows, :]
    for w in range(1, NW):
        e2n = e2n + parts_ref[w * n_rows:(w + 1) * n_rows, :]
    nid = nid_ref[...]
    pxnb = pxnb_ref[...] + b_n
    pgn_full = jnp.zeros_like(pxnb)
    for sg in range(n_glob):
        pgn_full = pgn_full + jnp.where(nid == sg, gproj_ref[1, sg], 0.0)
    node_out = pxnb + pgn_full + wn_a * e2n
    node_ref[...] = node_out
    lane = lax.broadcasted_iota(jnp.int32, (1, 128), 1)
    base = jnp.zeros((1, 128), jnp.float32)
    for sg in range(n_glob):
        m = nid == sg
        val = jnp.sum(jnp.where(m, pxnb, 0.0)) + gproj_ref[1, sg] * jnp.sum(
            m.astype(jnp.float32))
        base = base + jnp.where(lane == sg, val, 0.0)
    n2g = base + wn_a * jnp.sum(ngp_ref[...], axis=0, keepdims=True)
    e2g = jnp.sum(egp_ref[...], axis=0, keepdims=True)
    glob = (gproj_ref[2:3, 0:n_glob] + wg_n * n2g[:, 0:n_glob]
            + wg_e * e2g[:, 0:n_glob])
    glob_ref[...] = jnp.broadcast_to(glob, (8, n_glob))


def kernel(x, e, g, edges, node_idx, edge_idx,
           W_edge, b_edge, W_node, b_node, W_glob, b_glob):
    n, dn = x.shape
    ne, de = e.shape
    ng, dg = g.shape
    npad = ((n + 2047) // 2048) * 2048
    n_rows = npad // 128
    nvec = ne // LANES
    nv_lo = nvec // NW
    extra = nvec - nv_lo * NW
    eb_max = (nv_lo + (1 if extra else 0)) * LANES

    f32 = jnp.float32
    i32 = jnp.int32
    w_x = jnp.zeros((8, dn), f32)
    w_x = w_x.at[0].set(W_edge[0, de:de + dn])
    w_x = w_x.at[1].set(W_edge[0, de + dn:de + 2 * dn])
    w_x = w_x.at[2].set(W_node[0, :dn])
    w_e = jnp.zeros((8, de), f32).at[0].set(W_edge[0, :de])
    w_g = jnp.zeros((8, dg), f32)
    w_g = w_g.at[0].set(W_edge[0, de + 2 * dn:])
    w_g = w_g.at[1].set(W_node[0, dn:dn + dg])
    w_g = w_g.at[2].set(W_glob[0, :dg])
    b_g = jnp.zeros((8, 1), f32).at[0].set(b_edge[0]).at[2].set(b_glob[0])

    xblk = 2048
    ps, pd, pxn = pl.pallas_call(
        _xproj_body,
        grid=(npad // xblk,),
        in_specs=[
            pl.BlockSpec((8, dn), lambda i: (0, 0)),
            pl.BlockSpec((xblk, dn), lambda i: (i, 0)),
        ],
        out_specs=[
            pl.BlockSpec((xblk,), lambda i: (i,)),
            pl.BlockSpec((xblk,), lambda i: (i,)),
            pl.BlockSpec((xblk,), lambda i: (i,)),
        ],
        out_shape=[jax.ShapeDtypeStruct((npad,), f32)] * 3,
    )(w_x, x)

    eblk = 8192
    epad = ((ne + eblk - 1) // eblk) * eblk
    pe = pl.pallas_call(
        _eproj_body,
        grid=(epad // eblk,),
        in_specs=[
            pl.BlockSpec((8, de), lambda i: (0, 0)),
            pl.BlockSpec((eblk, de), lambda i: (i, 0)),
        ],
        out_specs=pl.BlockSpec((eblk,), lambda i: (i,)),
        out_shape=jax.ShapeDtypeStruct((epad,), f32),
    )(w_e, e)

    pg8 = pl.pallas_call(
        _gproj_body,
        out_shape=jax.ShapeDtypeStruct((8, ng), f32),
    )(w_g, b_g, g)

    nidp = jnp.full((npad,), ng, i32).at[:n].set(node_idx)
    pge128 = jnp.zeros((128,), f32).at[:ng].set(pg8[0])
    zeros_n = jnp.zeros((npad,), f32)

    mesh = plsc.VectorSubcoreMesh(core_axis_name="c", subcore_axis_name="s",
                                  num_cores=NC, num_subcores=NS)
    sc_edge = pl.kernel(
        functools.partial(_sc_edge_body, nv_lo=nv_lo, extra=extra),
        out_type=[
            jax.ShapeDtypeStruct((ne,), f32),
            jax.ShapeDtypeStruct((NW, npad), f32),
            jax.ShapeDtypeStruct((NW, 128), f32),
            jax.ShapeDtypeStruct((NW, 128), f32),
        ],
        mesh=mesh,
        compiler_params=pltpu.CompilerParams(needs_layout_passes=False),
        scratch_types=[
            pltpu.VMEM((npad,), f32),
            pltpu.VMEM((npad,), f32),
            pltpu.VMEM((npad,), i32),
            pltpu.VMEM((npad,), f32),
            pltpu.VMEM((128,), f32),
            pltpu.VMEM((128,), f32),
            pltpu.VMEM((128,), f32),
            pltpu.VMEM((eb_max,), i32),
            pltpu.VMEM((eb_max,), i32),
            pltpu.VMEM((eb_max,), i32),
            pltpu.VMEM((eb_max,), f32),
            pltpu.VMEM((eb_max,), f32),
        ],
    )
    eo, e2n_parts, eg_parts, ng_parts = sc_edge(
        ps, pd, pge128, nidp, pe, edges[0], edges[1], edge_idx, zeros_n)

    scal = jnp.stack([W_node[0, dn + dg], W_glob[0, dg], W_glob[0, dg + 1],
                      b_node[0]])
    node2, glob8 = pl.pallas_call(
        functools.partial(_finish_body, n_rows=n_rows, n_glob=ng),
        in_specs=[
            pl.BlockSpec(memory_space=pltpu.VMEM),
            pl.BlockSpec(memory_space=pltpu.VMEM),
            pl.BlockSpec(memory_space=pltpu.VMEM),
            pl.BlockSpec(memory_space=pltpu.VMEM),
            pl.BlockSpec(memory_space=pltpu.VMEM),
            pl.BlockSpec(memory_space=pltpu.VMEM),
            pl.BlockSpec(memory_space=pltpu.SMEM),
        ],
        out_shape=[
            jax.ShapeDtypeStruct((n_rows, 128), f32),
            jax.ShapeDtypeStruct((8, ng), f32),
        ],
    )(e2n_parts.reshape(NW * n_rows, 128), pxn.reshape(n_rows, 128),
      nidp.reshape(n_rows, 128), pg8, eg_parts, ng_parts, scal)

    edge_out = eo[:, None]
    node_out = node2.reshape(npad)[:n, None]
    glob_out = glob8[0][:, None]
    return (edge_out, node_out, glob_out)


# speedup vs baseline: 11.2521x; 1.3232x over previous
import functools

import jax
import jax.numpy as jnp
from jax import lax
from jax.experimental import pallas as pl
from jax.experimental.pallas import tpu as pltpu
from jax.experimental.pallas import tpu_sc as plsc

NC = 2
NS = 16
NW = NC * NS
LANES = 16


def _proj_body(wx_ref, we_ref, wg_ref, bg_ref, g_ref, x_ref, e_ref,
               px_ref, pe_ref, pg_ref):
    px_ref[...] = lax.dot_general(
        wx_ref[...], x_ref[...], (((1,), (1,)), ((), ())),
        preferred_element_type=jnp.float32,
        precision=lax.Precision.DEFAULT)
    pe_ref[...] = lax.dot_general(
        e_ref[...], we_ref[...], (((1,), (0,)), ((), ())),
        preferred_element_type=jnp.float32,
        precision=lax.Precision.DEFAULT)

    @pl.when(pl.program_id(0) == 0)
    def _():
        pg_ref[...] = lax.dot_general(
            wg_ref[...], g_ref[...], (((1,), (1,)), ((), ())),
            preferred_element_type=jnp.float32,
            precision=lax.Precision.DEFAULT) + bg_ref[...]


def _sc_stage_chunk(pe_hbm, src_hbm, dst_hbm, eidx_hbm,
                    src_v, dst_v, eidx_v, pe_v, base, eb):
    pltpu.sync_copy(src_hbm.at[pl.ds(base, eb)], src_v.at[pl.ds(0, eb)])
    pltpu.sync_copy(dst_hbm.at[pl.ds(base, eb)], dst_v.at[pl.ds(0, eb)])
    pltpu.sync_copy(eidx_hbm.at[pl.ds(base, eb)], eidx_v.at[pl.ds(0, eb)])
    pltpu.sync_copy(pe_hbm.at[pl.ds(base, eb)], pe_v.at[pl.ds(0, eb)])


def _sc_edge_body(ps_hbm, pd_hbm, pge_hbm, nid_hbm, pe_hbm, src_hbm, dst_hbm,
                  eidx_hbm, zeros_hbm,
                  eo_hbm, e2n_hbm, eg_hbm, ng_hbm,
                  ps_v, pd_v, nid_v, acc_v, pge_v, eg_v, ng_v,
                  src_v, dst_v, eidx_v, pe_v, eo_v,
                  nv_lo, extra):
    c = lax.axis_index("c")
    s = lax.axis_index("s")
    wid = s * NC + c
    is_hi = wid < extra
    base = jnp.where(is_hi, wid * (nv_lo + 1), wid * nv_lo + extra) * LANES
    pltpu.sync_copy(ps_hbm, ps_v)
    pltpu.sync_copy(pd_hbm, pd_v)
    pltpu.sync_copy(nid_hbm, nid_v)
    pltpu.sync_copy(pge_hbm, pge_v)
    pltpu.sync_copy(zeros_hbm, acc_v)
    eb_hi = (nv_lo + 1) * LANES
    eb_lo = nv_lo * LANES
    pl.when(is_hi)(lambda: _sc_stage_chunk(
        pe_hbm, src_hbm, dst_hbm, eidx_hbm,
        src_v, dst_v, eidx_v, pe_v, base, eb_hi))
    pl.when(jnp.logical_not(is_hi))(lambda: _sc_stage_chunk(
        pe_hbm, src_hbm, dst_hbm, eidx_hbm,
        src_v, dst_v, eidx_v, pe_v, base, eb_lo))
    zero16 = jnp.zeros((LANES,), jnp.float32)
    for i in range(0, 128, LANES):
        eg_v[pl.ds(i, LANES)] = zero16
        ng_v[pl.ds(i, LANES)] = zero16

    nvec = jnp.where(is_hi, nv_lo + 1, nv_lo)

    def body(j, carry):
        o = j * LANES
        s16 = src_v[pl.ds(o, LANES)]
        d16 = dst_v[pl.ds(o, LANES)]
        g16 = eidx_v[pl.ds(o, LANES)]
        val = (pe_v[pl.ds(o, LANES)]
               + plsc.load_gather(ps_v, [s16])
               + plsc.load_gather(pd_v, [d16])
               + plsc.load_gather(pge_v, [g16]))
        eo_v[pl.ds(o, LANES)] = val
        plsc.addupdate_scatter(acc_v, [d16], val)
        plsc.addupdate_scatter(eg_v, [g16], val)
        nd16 = plsc.load_gather(nid_v, [d16])
        plsc.addupdate_scatter(ng_v, [nd16], val)
        return carry

    lax.fori_loop(0, nvec, body, 0)
    pl.when(is_hi)(lambda: pltpu.sync_copy(
        eo_v.at[pl.ds(0, eb_hi)], eo_hbm.at[pl.ds(base, eb_hi)]))
    pl.when(jnp.logical_not(is_hi))(lambda: pltpu.sync_copy(
        eo_v.at[pl.ds(0, eb_lo)], eo_hbm.at[pl.ds(base, eb_lo)]))
    pltpu.sync_copy(acc_v, e2n_hbm.at[wid])
    pltpu.sync_copy(eg_v, eg_hbm.at[wid])
    pltpu.sync_copy(ng_v, ng_hbm.at[wid])


def _finish_body(parts_ref, pxnb_ref, nid_ref, gproj_ref, egp_ref, ngp_ref,
                 scal_ref, node_ref, glob_ref, n_rows, n_glob):
    wn_a = scal_ref[0]
    wg_n = scal_ref[1]
    wg_e = scal_ref[2]
    b_n = scal_ref[3]
    e2n = parts_ref[0:n_rows, :]
    for w in range(1, NW):
        e2n = e2n + parts_ref[w * n_rows:(w + 1) * n_rows, :]
    nid = nid_ref[...]
    pxnb = pxnb_ref[...] + b_n
    pgn_full = jnp.zeros_like(pxnb)
    for sg in range(n_glob):
        pgn_full = pgn_full + jnp.where(nid == sg, gproj_ref[1, sg], 0.0)
    node_out = pxnb + pgn_full + wn_a * e2n
    node_ref[...] = node_out
    lane = lax.broadcasted_iota(jnp.int32, (1, 128), 1)
    base = jnp.zeros((1, 128), jnp.float32)
    for sg in range(n_glob):
        m = nid == sg
        val = jnp.sum(jnp.where(m, pxnb, 0.0)) + gproj_ref[1, sg] * jnp.sum(
            m.astype(jnp.float32))
        base = base + jnp.where(lane == sg, val, 0.0)
    n2g = base + wn_a * jnp.sum(ngp_ref[...], axis=0, keepdims=True)
    e2g = jnp.sum(egp_ref[...], axis=0, keepdims=True)
    glob = (gproj_ref[2:3, 0:n_glob] + wg_n * n2g[:, 0:n_glob]
            + wg_e * e2g[:, 0:n_glob])
    glob_ref[...] = jnp.broadcast_to(glob, (8, n_glob))


def kernel(x, e, g, edges, node_idx, edge_idx,
           W_edge, b_edge, W_node, b_node, W_glob, b_glob):
    n, dn = x.shape
    ne, de = e.shape
    ng, dg = g.shape
    npad = ((n + 2047) // 2048) * 2048
    n_rows = npad // 128
    nvec = ne // LANES
    nv_lo = nvec // NW
    extra = nvec - nv_lo * NW
    eb_max = (nv_lo + (1 if extra else 0)) * LANES

    f32 = jnp.float32
    i32 = jnp.int32
    w_x = jnp.zeros((8, dn), f32)
    w_x = w_x.at[0].set(W_edge[0, de:de + dn])
    w_x = w_x.at[1].set(W_edge[0, de + dn:de + 2 * dn])
    w_x = w_x.at[2].set(W_node[0, :dn])
    epr = dn // de
    w_e16 = jnp.zeros((dn, epr), f32).at[
        jnp.arange(dn), jnp.arange(dn) // de].set(
        jnp.tile(W_edge[0, :de], epr))
    w_g = jnp.zeros((8, dg), f32)
    w_g = w_g.at[0].set(W_edge[0, de + 2 * dn:])
    w_g = w_g.at[1].set(W_node[0, dn:dn + dg])
    w_g = w_g.at[2].set(W_glob[0, :dg])
    b_g = jnp.zeros((8, 1), f32).at[0].set(b_edge[0]).at[2].set(b_glob[0])

    assert (ne * de) % dn == 0 and (ne * de) // dn == n
    e2d = e.reshape(n, dn)
    xblk = 2048
    px8, pe2d, pg8 = pl.pallas_call(
        _proj_body,
        grid=(npad // xblk,),
        in_specs=[
            pl.BlockSpec((8, dn), lambda i: (0, 0)),
            pl.BlockSpec((dn, epr), lambda i: (0, 0)),
            pl.BlockSpec((8, dg), lambda i: (0, 0)),
            pl.BlockSpec((8, 1), lambda i: (0, 0)),
            pl.BlockSpec((ng, dg), lambda i: (0, 0)),
            pl.BlockSpec((xblk, dn), lambda i: (i, 0)),
            pl.BlockSpec((xblk, dn), lambda i: (i, 0)),
        ],
        out_specs=[
            pl.BlockSpec((8, xblk), lambda i: (0, i)),
            pl.BlockSpec((xblk, epr), lambda i: (i, 0)),
            pl.BlockSpec((8, ng), lambda i: (0, 0)),
        ],
        out_shape=[
            jax.ShapeDtypeStruct((8, npad), f32),
            jax.ShapeDtypeStruct((npad, epr), f32),
            jax.ShapeDtypeStruct((8, ng), f32),
        ],
    )(w_x, w_e16, w_g, b_g, g, x, e2d)
    pe = pe2d[:n].reshape(ne)
    ps = px8[0]
    pd = px8[1]
    pxn = px8[2]

    nidp = jnp.full((npad,), ng, i32).at[:n].set(node_idx)
    pge128 = jnp.zeros((128,), f32).at[:ng].set(pg8[0])
    zeros_n = jnp.zeros((npad,), f32)

    mesh = plsc.VectorSubcoreMesh(core_axis_name="c", subcore_axis_name="s",
                                  num_cores=NC, num_subcores=NS)
    sc_edge = pl.kernel(
        functools.partial(_sc_edge_body, nv_lo=nv_lo, extra=extra),
        out_type=[
            jax.ShapeDtypeStruct((ne,), f32),
            jax.ShapeDtypeStruct((NW, npad), f32),
            jax.ShapeDtypeStruct((NW, 128), f32),
            jax.ShapeDtypeStruct((NW, 128), f32),
        ],
        mesh=mesh,
        compiler_params=pltpu.CompilerParams(needs_layout_passes=False),
        scratch_types=[
            pltpu.VMEM((npad,), f32),
            pltpu.VMEM((npad,), f32),
            pltpu.VMEM((npad,), i32),
            pltpu.VMEM((npad,), f32),
            pltpu.VMEM((128,), f32),
            pltpu.VMEM((128,), f32),
            pltpu.VMEM((128,), f32),
            pltpu.VMEM((eb_max,), i32),
            pltpu.VMEM((eb_max,), i32),
            pltpu.VMEM((eb_max,), i32),
            pltpu.VMEM((eb_max,), f32),
            pltpu.VMEM((eb_max,), f32),
        ],
    )
    eo, e2n_parts, eg_parts, ng_parts = sc_edge(
        ps, pd, pge128, nidp, pe, edges[0], edges[1], edge_idx, zeros_n)

    scal = jnp.stack([W_node[0, dn + dg], W_glob[0, dg], W_glob[0, dg + 1],
                      b_node[0]])
    node2, glob8 = pl.pallas_call(
        functools.partial(_finish_body, n_rows=n_rows, n_glob=ng),
        in_specs=[
            pl.BlockSpec(memory_space=pltpu.VMEM),
            pl.BlockSpec(memory_space=pltpu.VMEM),
            pl.BlockSpec(memory_space=pltpu.VMEM),
            pl.BlockSpec(memory_space=pltpu.VMEM),
            pl.BlockSpec(memory_space=pltpu.VMEM),
            pl.BlockSpec(memory_space=pltpu.VMEM),
            pl.BlockSpec(memory_space=pltpu.SMEM),
        ],
        out_shape=[
            jax.ShapeDtypeStruct((n_rows, 128), f32),
            jax.ShapeDtypeStruct((8, ng), f32),
        ],
    )(e2n_parts.reshape(NW * n_rows, 128), pxn.reshape(n_rows, 128),
      nidp.reshape(n_rows, 128), pg8, eg_parts, ng_parts, scal)

    edge_out = eo[:, None]
    node_out = node2.reshape(npad)[:n, None]
    glob_out = glob8[0][:, None]
    return (edge_out, node_out, glob_out)


# speedup vs baseline: 11.2934x; 1.0037x over previous
import functools

import jax
import jax.numpy as jnp
from jax import lax
from jax.experimental import pallas as pl
from jax.experimental.pallas import tpu as pltpu
from jax.experimental.pallas import tpu_sc as plsc

NC = 2
NS = 16
NW = NC * NS
LANES = 16


def _xproj_body(wx_ref, wg_ref, bg_ref, g_ref, x_ref, px_ref, pg_ref):
    px_ref[...] = lax.dot_general(
        wx_ref[...], x_ref[...], (((1,), (1,)), ((), ())),
        preferred_element_type=jnp.float32)

    @pl.when(pl.program_id(0) == 0)
    def _():
        pg_ref[...] = lax.dot_general(
            wg_ref[...], g_ref[...], (((1,), (1,)), ((), ())),
            preferred_element_type=jnp.float32) + bg_ref[...]


def _eproj_body(we_ref, e_ref, pe_ref):
    pe_ref[...] = lax.dot_general(
        we_ref[...], e_ref[...], (((1,), (1,)), ((), ())),
        preferred_element_type=jnp.float32)


def _sc_stage_chunk(pe_hbm, src_hbm, dst_hbm, eidx_hbm,
                    src_v, dst_v, eidx_v, pe_v, base, eb):
    pltpu.sync_copy(src_hbm.at[pl.ds(base, eb)], src_v.at[pl.ds(0, eb)])
    pltpu.sync_copy(dst_hbm.at[pl.ds(base, eb)], dst_v.at[pl.ds(0, eb)])
    pltpu.sync_copy(eidx_hbm.at[pl.ds(base, eb)], eidx_v.at[pl.ds(0, eb)])
    pltpu.sync_copy(pe_hbm.at[pl.ds(base, eb)], pe_v.at[pl.ds(0, eb)])


def _sc_edge_body(ps_hbm, pd_hbm, pge_hbm, nid_hbm, pe_hbm, src_hbm, dst_hbm,
                  eidx_hbm, zeros_hbm,
                  eo_hbm, e2n_hbm, eg_hbm, ng_hbm,
                  ps_v, pd_v, nid_v, acc_v, pge_v, eg_v, ng_v,
                  src_v, dst_v, eidx_v, pe_v, eo_v,
                  nv_lo, extra, n_rows):
    c = lax.axis_index("c")
    s = lax.axis_index("s")
    wid = s * NC + c
    is_hi = wid < extra
    base = jnp.where(is_hi, wid * (nv_lo + 1), wid * nv_lo + extra) * LANES
    pltpu.sync_copy(ps_hbm, ps_v)
    pltpu.sync_copy(pd_hbm, pd_v)
    pltpu.sync_copy(nid_hbm, nid_v)
    pltpu.sync_copy(pge_hbm, pge_v)
    pltpu.sync_copy(zeros_hbm, acc_v)
    eb_hi = (nv_lo + 1) * LANES
    eb_lo = nv_lo * LANES
    pl.when(is_hi)(lambda: _sc_stage_chunk(
        pe_hbm, src_hbm, dst_hbm, eidx_hbm,
        src_v, dst_v, eidx_v, pe_v, base, eb_hi))
    pl.when(jnp.logical_not(is_hi))(lambda: _sc_stage_chunk(
        pe_hbm, src_hbm, dst_hbm, eidx_hbm,
        src_v, dst_v, eidx_v, pe_v, base, eb_lo))
    zero16 = jnp.zeros((LANES,), jnp.float32)
    for i in range(0, 128, LANES):
        eg_v[pl.ds(i, LANES)] = zero16
        ng_v[pl.ds(i, LANES)] = zero16

    nvec = jnp.where(is_hi, nv_lo + 1, nv_lo)

    def body(j, carry):
        o = j * LANES
        s16 = src_v[pl.ds(o, LANES)]
        d16 = dst_v[pl.ds(o, LANES)]
        g16 = eidx_v[pl.ds(o, LANES)]
        val = (pe_v[pl.ds(o, LANES)]
               + plsc.load_gather(ps_v, [s16])
               + plsc.load_gather(pd_v, [d16])
               + plsc.load_gather(pge_v, [g16]))
        eo_v[pl.ds(o, LANES)] = val
        plsc.addupdate_scatter(acc_v, [d16 >> 7, d16 & 127], val)
        plsc.addupdate_scatter(eg_v, [g16], val)
        nd16 = plsc.load_gather(nid_v, [d16])
        plsc.addupdate_scatter(ng_v, [nd16], val)
        return carry

    lax.fori_loop(0, nvec, body, 0)
    pl.when(is_hi)(lambda: pltpu.sync_copy(
        eo_v.at[pl.ds(0, eb_hi)], eo_hbm.at[pl.ds(base, eb_hi)]))
    pl.when(jnp.logical_not(is_hi))(lambda: pltpu.sync_copy(
        eo_v.at[pl.ds(0, eb_lo)], eo_hbm.at[pl.ds(base, eb_lo)]))
    pltpu.sync_copy(acc_v, e2n_hbm.at[pl.ds(wid * n_rows, n_rows)])
    pltpu.sync_copy(eg_v, eg_hbm.at[wid])
    pltpu.sync_copy(ng_v, ng_hbm.at[wid])


def _finish_body(parts_ref, pxnb_ref, nid_ref, gproj_ref, egp_ref, ngp_ref,
                 scal_ref, node_ref, glob_ref, n_rows, n_glob):
    wn_a = scal_ref[0]
    wg_n = scal_ref[1]
    wg_e = scal_ref[2]
    b_n = scal_ref[3]
    e2n = parts_ref[0:n_rows, :]
    for w in range(1, NW):
        e2n = e2n + parts_ref[w * n_rows:(w + 1) * n_rows, :]
    nid = nid_ref[...]
    pxnb = pxnb_ref[...] + b_n
    pgn_full = jnp.zeros_like(pxnb)
    for sg in range(n_glob):
        pgn_full = pgn_full + jnp.where(nid == sg, gproj_ref[1, sg], 0.0)
    node_out = pxnb + pgn_full + wn_a * e2n
    node_ref[...] = node_out
    lane = lax.broadcasted_iota(jnp.int32, (1, 128), 1)
    base = jnp.zeros((1, 128), jnp.float32)
    for sg in range(n_glob):
        m = nid == sg
        val = jnp.sum(jnp.where(m, pxnb, 0.0)) + gproj_ref[1, sg] * jnp.sum(
            m.astype(jnp.float32))
        base = base + jnp.where(lane == sg, val, 0.0)
    n2g = base + wn_a * jnp.sum(ngp_ref[...], axis=0, keepdims=True)
    e2g = jnp.sum(egp_ref[...], axis=0, keepdims=True)
    glob = (gproj_ref[2:3, 0:n_glob] + wg_n * n2g[:, 0:n_glob]
            + wg_e * e2g[:, 0:n_glob])
    glob_ref[...] = jnp.broadcast_to(glob, (8, n_glob))


def kernel(x, e, g, edges, node_idx, edge_idx,
           W_edge, b_edge, W_node, b_node, W_glob, b_glob):
    n, dn = x.shape
    ne, de = e.shape
    ng, dg = g.shape
    npad = ((n + 2047) // 2048) * 2048
    n_rows = npad // 128
    nvec = ne // LANES
    nv_lo = nvec // NW
    extra = nvec - nv_lo * NW
    eb_max = (nv_lo + (1 if extra else 0)) * LANES

    f32 = jnp.float32
    i32 = jnp.int32
    w_x = jnp.zeros((8, dn), f32)
    w_x = w_x.at[0].set(W_edge[0, de:de + dn])
    w_x = w_x.at[1].set(W_edge[0, de + dn:de + 2 * dn])
    w_x = w_x.at[2].set(W_node[0, :dn])
    w_e = jnp.zeros((8, de), f32).at[0].set(W_edge[0, :de])
    w_g = jnp.zeros((8, dg), f32)
    w_g = w_g.at[0].set(W_edge[0, de + 2 * dn:])
    w_g = w_g.at[1].set(W_node[0, dn:dn + dg])
    w_g = w_g.at[2].set(W_glob[0, :dg])
    b_g = jnp.zeros((8, 1), f32).at[0].set(b_edge[0]).at[2].set(b_glob[0])

    xblk = 2048
    px8, pg8 = pl.pallas_call(
        _xproj_body,
        grid=(npad // xblk,),
        in_specs=[
            pl.BlockSpec((8, dn), lambda i: (0, 0)),
            pl.BlockSpec((8, dg), lambda i: (0, 0)),
            pl.BlockSpec((8, 1), lambda i: (0, 0)),
            pl.BlockSpec((ng, dg), lambda i: (0, 0)),
            pl.BlockSpec((xblk, dn), lambda i: (i, 0)),
        ],
        out_specs=[
            pl.BlockSpec((8, xblk), lambda i: (0, i)),
            pl.BlockSpec((8, ng), lambda i: (0, 0)),
        ],
        out_shape=[
            jax.ShapeDtypeStruct((8, npad), f32),
            jax.ShapeDtypeStruct((8, ng), f32),
        ],
    )(w_x, w_g, b_g, g, x)
    ps = px8[0]
    pd = px8[1]
    pxn = px8[2]

    eblk = 8192
    epad = ((ne + eblk - 1) // eblk) * eblk
    pe8 = pl.pallas_call(
        _eproj_body,
        grid=(epad // eblk,),
        in_specs=[
            pl.BlockSpec((8, de), lambda i: (0, 0)),
            pl.BlockSpec((eblk, de), lambda i: (i, 0)),
        ],
        out_specs=pl.BlockSpec((8, eblk), lambda i: (0, i)),
        out_shape=jax.ShapeDtypeStruct((8, epad), f32),
    )(w_e, e)
    pe = pe8[0]

    nidp = jnp.full((npad,), ng, i32).at[:n].set(node_idx)
    pge128 = jnp.zeros((128,), f32).at[:ng].set(pg8[0])
    zeros_n = jnp.zeros((n_rows, 128), f32)

    mesh = plsc.VectorSubcoreMesh(core_axis_name="c", subcore_axis_name="s",
                                  num_cores=NC, num_subcores=NS)
    sc_edge = pl.kernel(
        functools.partial(_sc_edge_body, nv_lo=nv_lo, extra=extra,
                          n_rows=n_rows),
        out_type=[
            jax.ShapeDtypeStruct((ne,), f32),
            jax.ShapeDtypeStruct((NW * n_rows, 128), f32),
            jax.ShapeDtypeStruct((NW, 128), f32),
            jax.ShapeDtypeStruct((NW, 128), f32),
        ],
        mesh=mesh,
        compiler_params=pltpu.CompilerParams(needs_layout_passes=False),
        scratch_types=[
            pltpu.VMEM((npad,), f32),
            pltpu.VMEM((npad,), f32),
            pltpu.VMEM((npad,), i32),
            pltpu.VMEM((n_rows, 128), f32),
            pltpu.VMEM((128,), f32),
            pltpu.VMEM((128,), f32),
            pltpu.VMEM((128,), f32),
            pltpu.VMEM((eb_max,), i32),
            pltpu.VMEM((eb_max,), i32),
            pltpu.VMEM((eb_max,), i32),
            pltpu.VMEM((eb_max,), f32),
            pltpu.VMEM((eb_max,), f32),
        ],
    )
    eo, e2n_parts, eg_parts, ng_parts = sc_edge(
        ps, pd, pge128, nidp, pe, edges[0], edges[1], edge_idx, zeros_n)

    scal = jnp.stack([W_node[0, dn + dg], W_glob[0, dg], W_glob[0, dg + 1],
                      b_node[0]])
    node2, glob8 = pl.pallas_call(
        functools.partial(_finish_body, n_rows=n_rows, n_glob=ng),
        in_specs=[
            pl.BlockSpec(memory_space=pltpu.VMEM),
            pl.BlockSpec(memory_space=pltpu.VMEM),
            pl.BlockSpec(memory_space=pltpu.VMEM),
            pl.BlockSpec(memory_space=pltpu.VMEM),
            pl.BlockSpec(memory_space=pltpu.VMEM),
            pl.BlockSpec(memory_space=pltpu.VMEM),
            pl.BlockSpec(memory_space=pltpu.SMEM),
        ],
        out_shape=[
            jax.ShapeDtypeStruct((n_rows, 128), f32),
            jax.ShapeDtypeStruct((8, ng), f32),
        ],
    )(e2n_parts, pxn.reshape(n_rows, 128), nidp.reshape(n_rows, 128),
      pg8, eg_parts, ng_parts, scal)

    edge_out = eo[:, None]
    node_out = node2.reshape(npad)[:n, None]
    glob_out = glob8[0][:, None]
    return (edge_out, node_out, glob_out)


# speedup vs baseline: 12.4094x; 1.0988x over previous
import functools

import jax
import jax.numpy as jnp
from jax import lax
from jax.experimental import pallas as pl
from jax.experimental.pallas import tpu as pltpu
from jax.experimental.pallas import tpu_sc as plsc

NC = 2
NS = 16
NW = NC * NS
LANES = 16


def _xproj_body(wx_ref, wg_ref, bg_ref, g_ref, x_ref, px_ref, pg_ref):
    px_ref[...] = lax.dot_general(
        wx_ref[...], x_ref[...], (((1,), (1,)), ((), ())),
        preferred_element_type=jnp.float32)

    @pl.when(pl.program_id(0) == 0)
    def _():
        pg_ref[...] = lax.dot_general(
            wg_ref[...], g_ref[...], (((1,), (1,)), ((), ())),
            preferred_element_type=jnp.float32) + bg_ref[...]


def _sc_edge_body(ps_hbm, pd_hbm, pge_hbm, we_hbm, e_hbm, src_hbm, dst_hbm,
                  eidx_hbm, zeros_hbm,
                  eo_hbm, e2n_hbm, eg_hbm,
                  ps_v, pd_v, acc_v, pge_v, we_v, eg_v,
                  src_v, dst_v, eidx_v, eo_v, e_v,
                  nv_lo, extra, n_rows, de):
    c = lax.axis_index("c")
    s = lax.axis_index("s")
    wid = s * NC + c
    is_hi = wid < extra
    base = jnp.where(is_hi, wid * (nv_lo + 1), wid * nv_lo + extra) * LANES
    nv0 = nv_lo // 2
    eb0 = nv0 * LANES
    nv1_hi = nv_lo + 1 - nv0
    nv1_lo = nv_lo - nv0
    eb_hi = (nv_lo + 1) * LANES
    eb_lo = nv_lo * LANES
    pltpu.sync_copy(ps_hbm, ps_v)
    pltpu.sync_copy(pd_hbm, pd_v)
    pltpu.sync_copy(pge_hbm, pge_v)
    pltpu.sync_copy(we_hbm, we_v)
    pltpu.sync_copy(zeros_hbm, acc_v)
    pl.when(is_hi)(lambda: _stage(src_hbm, dst_hbm, eidx_hbm,
                                  src_v, dst_v, eidx_v, base, eb_hi))
    pl.when(jnp.logical_not(is_hi))(
        lambda: _stage(src_hbm, dst_hbm, eidx_hbm,
                       src_v, dst_v, eidx_v, base, eb_lo))
    zero16 = jnp.zeros((LANES,), jnp.float32)
    for i in range(0, 128, LANES):
        eg_v[pl.ds(i, LANES)] = zero16

    iota = lax.iota(jnp.int32, LANES)
    il16 = iota * de
    wvec = we_v[pl.ds(0, LANES)]
    wes = [wvec[d] for d in range(de)]

    def body_at(off_vec):
        def body(j, carry):
            o = j * LANES
            go = off_vec * LANES + o
            s16 = src_v[pl.ds(go, LANES)]
            d16 = dst_v[pl.ds(go, LANES)]
            g16 = eidx_v[pl.ds(go, LANES)]
            rb16 = il16 + o * de
            val = (plsc.load_gather(ps_v, [s16])
                   + plsc.load_gather(pd_v, [d16])
                   + plsc.load_gather(pge_v, [g16]))
            for d in range(de):
                val = val + wes[d] * plsc.load_gather(e_v, [rb16 + d])
            eo_v[pl.ds(go, LANES)] = val
            plsc.addupdate_scatter(acc_v, [d16 >> 7, d16 & 127], val)
            plsc.addupdate_scatter(eg_v, [g16], val)
            return carry
        return body

    pltpu.sync_copy(e_hbm.at[pl.ds(base * de, eb0 * de)],
                    e_v.at[pl.ds(0, eb0 * de)])
    lax.fori_loop(0, nv0, body_at(0), 0)
    pl.when(is_hi)(lambda: pltpu.sync_copy(
        e_hbm.at[pl.ds((base + eb0) * de, (eb_hi - eb0) * de)],
        e_v.at[pl.ds(0, (eb_hi - eb0) * de)]))
    pl.when(jnp.logical_not(is_hi))(lambda: pltpu.sync_copy(
        e_hbm.at[pl.ds((base + eb0) * de, (eb_lo - eb0) * de)],
        e_v.at[pl.ds(0, (eb_lo - eb0) * de)]))
    nv1 = jnp.where(is_hi, nv1_hi, nv1_lo)
    lax.fori_loop(0, nv1, body_at(nv0), 0)

    pl.when(is_hi)(lambda: pltpu.sync_copy(
        eo_v.at[pl.ds(0, eb_hi)], eo_hbm.at[pl.ds(base, eb_hi)]))
    pl.when(jnp.logical_not(is_hi))(lambda: pltpu.sync_copy(
        eo_v.at[pl.ds(0, eb_lo)], eo_hbm.at[pl.ds(base, eb_lo)]))
    pltpu.sync_copy(acc_v, e2n_hbm.at[pl.ds(wid * n_rows, n_rows)])
    pltpu.sync_copy(eg_v, eg_hbm.at[wid])


def _stage(src_hbm, dst_hbm, eidx_hbm, src_v, dst_v, eidx_v, base, eb):
    pltpu.sync_copy(src_hbm.at[pl.ds(base, eb)], src_v.at[pl.ds(0, eb)])
    pltpu.sync_copy(dst_hbm.at[pl.ds(base, eb)], dst_v.at[pl.ds(0, eb)])
    pltpu.sync_copy(eidx_hbm.at[pl.ds(base, eb)], eidx_v.at[pl.ds(0, eb)])


def _finish_body(parts_ref, pxnb_ref, nid_ref, gproj_ref, egp_ref,
                 scal_ref, node_ref, glob_ref, n_rows, n_glob):
    wn_a = scal_ref[0]
    wg_n = scal_ref[1]
    wg_e = scal_ref[2]
    b_n = scal_ref[3]
    e2n = parts_ref[0:n_rows, :]
    for w in range(1, NW):
        e2n = e2n + parts_ref[w * n_rows:(w + 1) * n_rows, :]
    nid = nid_ref[...]
    pxnb = pxnb_ref[...] + b_n
    pgn_full = jnp.zeros_like(pxnb)
    for sg in range(n_glob):
        pgn_full = pgn_full + jnp.where(nid == sg, gproj_ref[1, sg], 0.0)
    node_out = pxnb + pgn_full + wn_a * e2n
    node_ref[...] = node_out
    lane = lax.broadcasted_iota(jnp.int32, (1, 128), 1)
    n2g = jnp.zeros((1, 128), jnp.float32)
    for sg in range(n_glob):
        val = jnp.sum(jnp.where(nid == sg, node_out, 0.0))
        n2g = n2g + jnp.where(lane == sg, val, 0.0)
    e2g = jnp.sum(egp_ref[...], axis=0, keepdims=True)
    glob = (gproj_ref[2:3, 0:n_glob] + wg_n * n2g[:, 0:n_glob]
            + wg_e * e2g[:, 0:n_glob])
    glob_ref[...] = jnp.broadcast_to(glob, (8, n_glob))


def kernel(x, e, g, edges, node_idx, edge_idx,
           W_edge, b_edge, W_node, b_node, W_glob, b_glob):
    n, dn = x.shape
    ne, de = e.shape
    ng, dg = g.shape
    npad = ((n + 2047) // 2048) * 2048
    n_rows = npad // 128
    nvec = ne // LANES
    nv_lo = nvec // NW
    extra = nvec - nv_lo * NW
    eb_max = (nv_lo + (1 if extra else 0)) * LANES
    e_half = (nv_lo + 1 - nv_lo // 2) * LANES

    f32 = jnp.float32
    i32 = jnp.int32
    w_x = jnp.zeros((8, dn), f32)
    w_x = w_x.at[0].set(W_edge[0, de:de + dn])
    w_x = w_x.at[1].set(W_edge[0, de + dn:de + 2 * dn])
    w_x = w_x.at[2].set(W_node[0, :dn])
    w_g = jnp.zeros((8, dg), f32)
    w_g = w_g.at[0].set(W_edge[0, de + 2 * dn:])
    w_g = w_g.at[1].set(W_node[0, dn:dn + dg])
    w_g = w_g.at[2].set(W_glob[0, :dg])
    b_g = jnp.zeros((8, 1), f32).at[0].set(b_edge[0]).at[2].set(b_glob[0])

    xblk = 2048
    px8, pg8 = pl.pallas_call(
        _xproj_body,
        grid=(npad // xblk,),
        in_specs=[
            pl.BlockSpec((8, dn), lambda i: (0, 0)),
            pl.BlockSpec((8, dg), lambda i: (0, 0)),
            pl.BlockSpec((8, 1), lambda i: (0, 0)),
            pl.BlockSpec((ng, dg), lambda i: (0, 0)),
            pl.BlockSpec((xblk, dn), lambda i: (i, 0)),
        ],
        out_specs=[
            pl.BlockSpec((8, xblk), lambda i: (0, i)),
            pl.BlockSpec((8, ng), lambda i: (0, 0)),
        ],
        out_shape=[
            jax.ShapeDtypeStruct((8, npad), f32),
            jax.ShapeDtypeStruct((8, ng), f32),
        ],
    )(w_x, w_g, b_g, g, x)
    ps = px8[0]
    pd = px8[1]
    pxn = px8[2]

    nidp = jnp.full((npad,), ng, i32).at[:n].set(node_idx)
    pge128 = jnp.zeros((128,), f32).at[:ng].set(pg8[0])
    we16 = jnp.zeros((128,), f32).at[:de].set(W_edge[0, :de])
    zeros_n = jnp.zeros((n_rows, 128), f32)

    mesh = plsc.VectorSubcoreMesh(core_axis_name="c", subcore_axis_name="s",
                                  num_cores=NC, num_subcores=NS)
    sc_edge = pl.kernel(
        functools.partial(_sc_edge_body, nv_lo=nv_lo, extra=extra,
                          n_rows=n_rows, de=de),
        out_type=[
            jax.ShapeDtypeStruct((ne,), f32),
            jax.ShapeDtypeStruct((NW * n_rows, 128), f32),
            jax.ShapeDtypeStruct((NW, 128), f32),
        ],
        mesh=mesh,
        compiler_params=pltpu.CompilerParams(needs_layout_passes=False),
        scratch_types=[
            pltpu.VMEM((npad,), f32),
            pltpu.VMEM((npad,), f32),
            pltpu.VMEM((n_rows, 128), f32),
            pltpu.VMEM((128,), f32),
            pltpu.VMEM((128,), f32),
            pltpu.VMEM((128,), f32),
            pltpu.VMEM((eb_max,), i32),
            pltpu.VMEM((eb_max,), i32),
            pltpu.VMEM((eb_max,), i32),
            pltpu.VMEM((eb_max,), f32),
            pltpu.VMEM((e_half * de,), f32),
        ],
    )
    eo, e2n_parts, eg_parts = sc_edge(
        ps, pd, pge128, we16, e.reshape(ne * de), edges[0], edges[1],
        edge_idx, zeros_n)

    scal = jnp.stack([W_node[0, dn + dg], W_glob[0, dg], W_glob[0, dg + 1],
                      b_node[0]])
    node2, glob8 = pl.pallas_call(
        functools.partial(_finish_body, n_rows=n_rows, n_glob=ng),
        in_specs=[
            pl.BlockSpec(memory_space=pltpu.VMEM),
            pl.BlockSpec(memory_space=pltpu.VMEM),
            pl.BlockSpec(memory_space=pltpu.VMEM),
            pl.BlockSpec(memory_space=pltpu.VMEM),
            pl.BlockSpec(memory_space=pltpu.VMEM),
            pl.BlockSpec(memory_space=pltpu.SMEM),
        ],
        out_shape=[
            jax.ShapeDtypeStruct((n_rows, 128), f32),
            jax.ShapeDtypeStruct((8, ng), f32),
        ],
    )(e2n_parts, pxn.reshape(n_rows, 128), nidp.reshape(n_rows, 128),
      pg8, eg_parts, scal)

    edge_out = eo[:, None]
    node_out = node2.reshape(npad)[:n, None]
    glob_out = glob8[0][:, None]
    return (edge_out, node_out, glob_out)
